# baseline (speedup 1.0000x reference)
import functools

import jax
import jax.numpy as jnp
from jax import lax
from jax.experimental import pallas as pl
from jax.experimental.pallas import tpu as pltpu
from jax.experimental.pallas import tpu_sc as plsc

N = 10000
E = 320000
F_IN = 128
H = 256
HH = 128
C = 16
G = 64
G_PAD = 72

N_PAD = 10240
NT = 16
ECH = 128
NCH = 162
EPT = NCH * ECH
E_PAD = EPT * NT
SCH = 2
SPB = 9
NOB = NCH // (SCH * SPB)
NPT = N_PAD // NT
ERW = 64
NEC = NPT // ERW

_mesh = plsc.VectorSubcoreMesh(core_axis_name="c", subcore_axis_name="s")


def _zero_rows(buf, nrows):
    zero = jnp.zeros((16,), jnp.float32)

    def body(e, _):
        for r in range(HH // 16):
            buf[e, pl.ds(r * 16, 16)] = zero
        return 0

    lax.fori_loop(0, nrows, body, 0)


def _deg_body(dst3d, ew_both, deg_out, deg_sp, dst_v, ew_v, buf_v):
    c = lax.axis_index("c")
    s = lax.axis_index("s")

    def zb(e, _):
        buf_v[pl.ds(e * 16, 16)] = jnp.zeros((16,), jnp.float32)
        return 0

    lax.fori_loop(0, NPT // 16, zb, 0)
    pltpu.sync_copy(buf_v, deg_sp.at[pl.ds(s * NPT, NPT)])
    plsc.subcore_barrier()

    pltpu.sync_copy(dst3d.at[s], dst_v)
    pltpu.sync_copy(ew_both.at[c, s], ew_v)

    def chunk(j, _):
        pltpu.sync_copy(ew_v.at[j], deg_sp.at[dst_v.at[j]], add=True)
        return 0

    lax.fori_loop(0, NCH, chunk, 0)
    plsc.subcore_barrier()

    pltpu.sync_copy(deg_sp.at[pl.ds(s * NPT, NPT)], buf_v)
    pltpu.sync_copy(buf_v, deg_out.at[c, pl.ds(s * NPT, NPT)])


_deg_kernel = pl.kernel(
    _deg_body,
    out_type=jax.ShapeDtypeStruct((2, N_PAD), jnp.float32),
    mesh=_mesh,
    scratch_types=[
        pltpu.VMEM_SHARED((N_PAD,), jnp.float32),
        pltpu.VMEM((NCH, ECH), jnp.int32),
        pltpu.VMEM((NCH, ECH), jnp.float32),
        pltpu.VMEM((NPT,), jnp.float32),
    ],
)


def _mm1a_body(x_ref, w_ref, xw_ref):
    xw_ref[...] = jnp.dot(
        x_ref[...], w_ref[...], preferred_element_type=jnp.float32)


def _mm1b_body(xwu_ref, deg_ref, xw_ref, dis_ref):
    deg = deg_ref[...]
    dis = jnp.where(deg > 0, lax.rsqrt(jnp.maximum(deg, 1e-12)), 0.0)
    dis_ref[...] = dis
    xw = xwu_ref[...] * dis[:N]
    for j in range(2):
        xw_ref[j, pl.ds(0, N)] = xw[:, j * HH:(j + 1) * HH]
        xw_ref[j, pl.ds(N, N_PAD - N)] = jnp.zeros((N_PAD - N, HH), jnp.float32)


def _mm1(x, W1, deg1):
    xwu = pl.pallas_call(
        _mm1a_body,
        out_shape=jax.ShapeDtypeStruct((N, H), jnp.float32),
    )(x, W1)
    return pl.pallas_call(
        _mm1b_body,
        out_shape=[
            jax.ShapeDtypeStruct((2, N_PAD, HH), jnp.float32),
            jax.ShapeDtypeStruct((N_PAD, 1), jnp.float32),
        ],
    )(xwu, deg1)


def _mm2_body(h_ref, w_ref, deg_ref, xw_ref, dis_ref):
    deg = deg_ref[...]
    dis = jnp.where(deg > 0, lax.rsqrt(jnp.maximum(deg, 1e-12)), 0.0)
    dis_ref[...] = dis
    w = w_ref[...]
    xw = jnp.dot(h_ref[0], w[:HH], preferred_element_type=jnp.float32)
    xw = xw + jnp.dot(h_ref[1], w[HH:], preferred_element_type=jnp.float32)
    xw = xw * dis
    for j in range(2):
        xw_ref[j] = xw[:, j * HH:(j + 1) * HH]


def _mm2(h, W2, deg2):
    return pl.pallas_call(
        _mm2_body,
        out_shape=[
            jax.ShapeDtypeStruct((2, N_PAD, HH), jnp.float32),
            jax.ShapeDtypeStruct((N_PAD, 1), jnp.float32),
        ],
    )(h, W2, deg2)


def _layer_body(scale, pool, *refs):
    if scale and pool:
        raise NotImplementedError
    if scale:
        (xw_hbm, src4d, dst4d, w4d, dis3d, bias2, h_out,
         acc_sp, src_v, dst_v, w_v, rows_a, rows_b, idx_a, idx_b,
         sidx_a, sidx_b, dis_v, bias_v, gsem_a, gsem_b, ssem_a, ssem_b) = refs
        pooled_sp = batch_v = pooled_out = None
    else:
        (xw_hbm, src4d, dst4d, dis3d, bias2, batch3d, pooled_out,
         acc_sp, pooled_sp, src_v, dst_v, rows_a, rows_b, idx_a, idx_b,
         sidx_a, sidx_b, dis_v, bias_v, batch_v,
         gsem_a, gsem_b, ssem_a, ssem_b) = refs
        w_v = None

    c = lax.axis_index("c")
    s = lax.axis_index("s")
    hbuf = rows_a.at[pl.ds(0, ERW)]
    bufs = ((rows_a, idx_a, sidx_a, gsem_a, ssem_a),
            (rows_b, idx_b, sidx_b, gsem_b, ssem_b))
    dummy_src = xw_hbm.at[pl.ds(0, ECH)]

    _zero_rows(rows_a, ECH)
    _zero_rows(rows_b, ECH)
    for jj in range(NEC):
        pltpu.sync_copy(hbuf, acc_sp.at[pl.ds(s * NPT + jj * ERW, ERW)])
    if pool:
        @pl.when(s < G_PAD // 8)
        def _():
            pltpu.sync_copy(rows_a.at[pl.ds(0, 8)], pooled_sp.at[pl.ds(s * 8, 8)])
    plsc.subcore_barrier()

    pltpu.sync_copy(dis3d.at[s], dis_v)
    pltpu.sync_copy(bias2.at[c], bias_v)
    if pool:
        pltpu.sync_copy(batch3d.at[s], batch_v)

    row_base = c * N_PAD

    for rows, _idx, sidx, _gs, ssem in bufs:
        for r in range(ECH // 16):
            sidx[pl.ds(r * 16, 16)] = lax.iota(jnp.int32, 16) + (s * NPT + r * 16)
        pltpu.async_copy(rows, acc_sp.at[sidx], ssem, add=True)

    def block(o, _):
        pltpu.sync_copy(src4d.at[s, pl.ds(o * SPB, SPB)], src_v)
        pltpu.sync_copy(dst4d.at[s, pl.ds(o * SPB, SPB)], dst_v)
        if scale:
            pltpu.sync_copy(w4d.at[s, pl.ds(o * SPB, SPB)], w_v)

        def super_chunk(gg, _):
            gd = []
            for k, (rows, idx, sidx, gsem, ssem) in enumerate(bufs):
                pltpu.make_async_copy(dummy_src, rows, ssem).wait()
                for r in range(ECH // 16):
                    idx[pl.ds(r * 16, 16)] = (
                        src_v[gg, k, pl.ds(r * 16, 16)] + row_base)
                gd.append(pltpu.async_copy(xw_hbm.at[idx], rows, gsem))
            for k, (rows, idx, sidx, gsem, ssem) in enumerate(bufs):
                gd[k].wait()
                if scale:
                    def sc16(ee, _, k=k, rows=rows):
                        wv = w_v[gg, k, pl.ds(ee * 16, 16)]
                        for kk in range(16):
                            e = ee * 16 + kk
                            w = wv[kk]
                            for r in range(HH // 16):
                                rows[e, pl.ds(r * 16, 16)] = (
                                    rows[e, pl.ds(r * 16, 16)] * w)
                        return 0

                    lax.fori_loop(0, ECH // 16, sc16, 0)
                for r in range(ECH // 16):
                    sidx[pl.ds(r * 16, 16)] = dst_v[gg, k, pl.ds(r * 16, 16)]
                pltpu.async_copy(rows, acc_sp.at[sidx], ssem, add=True)
            return 0

        lax.fori_loop(0, SPB, super_chunk, 0)
        return 0

    lax.fori_loop(0, NOB, block, 0)
    for rows, _idx, _sidx, _gs, ssem in bufs:
        pltpu.make_async_copy(dummy_src, rows, ssem).wait()
    plsc.subcore_barrier()

    def epi(jj, _):
        pltpu.sync_copy(acc_sp.at[pl.ds(s * NPT + jj * ERW, ERW)], hbuf)

        def row16(ee, _):
            dv = dis_v[jj, pl.ds(ee * 16, 16)]
            for k in range(16):
                e = ee * 16 + k
                d = dv[k]
                for r in range(HH // 16):
                    v = rows_a[e, pl.ds(r * 16, 16)] * d + bias_v[pl.ds(r * 16, 16)]
                    rows_a[e, pl.ds(r * 16, 16)] = jnp.maximum(v, 0.0)
            return 0

        lax.fori_loop(0, ERW // 16, row16, 0)
        if pool:
            pltpu.sync_copy(hbuf, pooled_sp.at[batch_v.at[jj]], add=True)
        else:
            pltpu.sync_copy(hbuf, h_out.at[c, pl.ds(s * NPT + jj * ERW, ERW)])
        return 0

    lax.fori_loop(0, NEC, epi, 0)

    if pool:
        plsc.subcore_barrier()

        @pl.when(s < G // 8)
        def _():
            pltpu.sync_copy(pooled_sp.at[pl.ds(s * 8, 8)], rows_a.at[pl.ds(0, 8)])
            pltpu.sync_copy(rows_a.at[pl.ds(0, 8)], pooled_out.at[c, pl.ds(s * 8, 8)])


_layer1_kernel = pl.kernel(
    functools.partial(_layer_body, True, False),
    out_type=jax.ShapeDtypeStruct((2, N_PAD, HH), jnp.float32),
    mesh=_mesh,
    scratch_types=[
        pltpu.VMEM_SHARED((N_PAD, HH), jnp.float32),
        pltpu.VMEM((SPB, SCH, ECH), jnp.int32),
        pltpu.VMEM((SPB, SCH, ECH), jnp.int32),
        pltpu.VMEM((SPB, SCH, ECH), jnp.float32),
        pltpu.VMEM((ECH, HH), jnp.float32),
        pltpu.VMEM((ECH, HH), jnp.float32),
        pltpu.VMEM((ECH,), jnp.int32),
        pltpu.VMEM((ECH,), jnp.int32),
        pltpu.VMEM((ECH,), jnp.int32),
        pltpu.VMEM((ECH,), jnp.int32),
        pltpu.VMEM((NEC, ERW), jnp.float32),
        pltpu.VMEM((HH,), jnp.float32),
        pltpu.SemaphoreType.DMA,
        pltpu.SemaphoreType.DMA,
        pltpu.SemaphoreType.DMA,
        pltpu.SemaphoreType.DMA,
    ],
)

_layer2_kernel = pl.kernel(
    functools.partial(_layer_body, False, True),
    out_type=jax.ShapeDtypeStruct((2, G, HH), jnp.float32),
    mesh=_mesh,
    scratch_types=[
        pltpu.VMEM_SHARED((N_PAD, HH), jnp.float32),
        pltpu.VMEM_SHARED((G_PAD, HH), jnp.float32),
        pltpu.VMEM((SPB, SCH, ECH), jnp.int32),
        pltpu.VMEM((SPB, SCH, ECH), jnp.int32),
        pltpu.VMEM((ECH, HH), jnp.float32),
        pltpu.VMEM((ECH, HH), jnp.float32),
        pltpu.VMEM((ECH,), jnp.int32),
        pltpu.VMEM((ECH,), jnp.int32),
        pltpu.VMEM((ECH,), jnp.int32),
        pltpu.VMEM((ECH,), jnp.int32),
        pltpu.VMEM((NEC, ERW), jnp.float32),
        pltpu.VMEM((HH,), jnp.float32),
        pltpu.VMEM((NEC, ERW), jnp.int32),
        pltpu.SemaphoreType.DMA,
        pltpu.SemaphoreType.DMA,
        pltpu.SemaphoreType.DMA,
        pltpu.SemaphoreType.DMA,
    ],
)


def _final_body(pooled_ref, batch_ref, wfc_ref, bfc_ref, out_ref):
    b2d = batch_ref[...]
    gids = lax.broadcasted_iota(jnp.int32, (G, N_PAD // 128, 128), 0)
    eq = (b2d[None, :, :] == gids).astype(jnp.float32)
    cnt = jnp.sum(eq, axis=(1, 2))
    cnt = jnp.maximum(cnt, 1.0)[:, None]
    pa = pooled_ref[0] / cnt
    pb = pooled_ref[1] / cnt
    wfc = wfc_ref[...]
    z = jnp.dot(pa, wfc[:HH], preferred_element_type=jnp.float32)
    z = z + jnp.dot(pb, wfc[HH:], preferred_element_type=jnp.float32)
    z = z + bfc_ref[...]
    m = jnp.max(z, axis=1, keepdims=True)
    e = jnp.exp(z - m)
    out_ref[...] = z - m - jnp.log(jnp.sum(e, axis=1, keepdims=True))


def _final(pooled, batch2d, Wfc, bfc):
    return pl.pallas_call(
        _final_body,
        out_shape=jax.ShapeDtypeStruct((G, C), jnp.float32),
    )(pooled, batch2d, Wfc, bfc.reshape(1, C))


def kernel(x, edge_index, edge_weight, batch, W1, b1, W2, b2, Wfc, bfc):
    f32 = jnp.float32
    i32 = jnp.int32
    loop = jnp.arange(N_PAD, dtype=i32)
    extra = E_PAD - E - N_PAD
    tail = jnp.full((extra,), N_PAD - 1, i32)
    sl_w = (loop < N).astype(f32)
    zw = jnp.zeros((extra,), f32)

    src_flat = jnp.concatenate([edge_index[0], loop, tail])
    dst_flat = jnp.concatenate([edge_index[1], loop, tail])
    w1_flat = jnp.concatenate([edge_weight, sl_w, zw])
    w2_flat = jnp.concatenate([jnp.ones((E,), f32), sl_w, zw])
    src4d = src_flat.reshape(NT, NOB * SPB, SCH, ECH)
    dst4d = dst_flat.reshape(NT, NOB * SPB, SCH, ECH)
    w1_4d = w1_flat.reshape(NT, NOB * SPB, SCH, ECH)
    dst3d = dst_flat.reshape(NT, NCH, ECH)
    ew_both = jnp.stack([w1_flat.reshape(NT, NCH, ECH),
                         w2_flat.reshape(NT, NCH, ECH)])

    batch_pad = jnp.concatenate(
        [batch.astype(i32), jnp.full((N_PAD - N,), G, i32)])
    batch3d = batch_pad.reshape(NT, NEC, ERW)
    batch2d = batch_pad.reshape(N_PAD // 128, 128)

    deg_both = _deg_kernel(dst3d, ew_both)
    deg1 = deg_both[0].reshape(N_PAD, 1)
    deg2 = deg_both[1].reshape(N_PAD, 1)

    xw1, dis1 = _mm1(x, W1, deg1)
    h1 = _layer1_kernel(
        xw1.reshape(2 * N_PAD, HH), src4d, dst4d, w1_4d,
        dis1.reshape(NT, NEC, ERW), b1.reshape(2, HH))

    xw2, dis2 = _mm2(h1, W2, deg2)
    pooled = _layer2_kernel(
        xw2.reshape(2 * N_PAD, HH), src4d, dst4d,
        dis2.reshape(NT, NEC, ERW), b2.reshape(2, HH), batch3d)

    return _final(pooled, batch2d, Wfc, bfc)


# speedup vs baseline: 1.0316x; 1.0316x over previous
import functools

import jax
import jax.numpy as jnp
from jax import lax
from jax.experimental import pallas as pl
from jax.experimental.pallas import tpu as pltpu
from jax.experimental.pallas import tpu_sc as plsc

N = 10000
E = 320000
F_IN = 128
H = 256
HH = 128
C = 16
G = 64
G_PAD = 72

N_PAD = 10240
NT = 16
ECH = 128
NCH = 162
EPT = NCH * ECH
E_PAD = EPT * NT
SCH = 2
SPB = 9
NOB = NCH // (SCH * SPB)
NPT = N_PAD // NT
ERW = 64
NEC = NPT // ERW

_mesh = plsc.VectorSubcoreMesh(core_axis_name="c", subcore_axis_name="s")


def _zero_rows(buf, nrows):
    zero = jnp.zeros((16,), jnp.float32)

    def body(e, _):
        for r in range(HH // 16):
            buf[e, pl.ds(r * 16, 16)] = zero
        return 0

    lax.fori_loop(0, nrows, body, 0)


def _deg_body(dst3d, ew_both, deg_out, deg_sp, dst_v, ew_v, buf_v):
    c = lax.axis_index("c")
    s = lax.axis_index("s")

    def zb(e, _):
        buf_v[pl.ds(e * 16, 16)] = jnp.zeros((16,), jnp.float32)
        return 0

    lax.fori_loop(0, NPT // 16, zb, 0)
    pltpu.sync_copy(buf_v, deg_sp.at[pl.ds(s * NPT, NPT)])
    plsc.subcore_barrier()

    pltpu.sync_copy(dst3d.at[s], dst_v)
    pltpu.sync_copy(ew_both.at[c, s], ew_v)

    def chunk(j, _):
        pltpu.sync_copy(ew_v.at[j], deg_sp.at[dst_v.at[j]], add=True)
        return 0

    lax.fori_loop(0, NCH, chunk, 0)
    plsc.subcore_barrier()

    pltpu.sync_copy(deg_sp.at[pl.ds(s * NPT, NPT)], buf_v)
    pltpu.sync_copy(buf_v, deg_out.at[c, pl.ds(s * NPT, NPT)])


_deg_kernel = pl.kernel(
    _deg_body,
    out_type=jax.ShapeDtypeStruct((2, N_PAD), jnp.float32),
    mesh=_mesh,
    scratch_types=[
        pltpu.VMEM_SHARED((N_PAD,), jnp.float32),
        pltpu.VMEM((NCH, ECH), jnp.int32),
        pltpu.VMEM((NCH, ECH), jnp.float32),
        pltpu.VMEM((NPT,), jnp.float32),
    ],
)


def _mm1_body(x_ref, w_ref, deg_ref, xw_ref, dis_ref):
    deg = deg_ref[...]
    dis = jnp.where(deg > 0, lax.rsqrt(jnp.maximum(deg, 1e-12)), 0.0)
    dis_ref[...] = dis
    xw = jnp.dot(x_ref[...], w_ref[...], preferred_element_type=jnp.float32)
    xw = xw * dis[:N]
    for j in range(2):
        xw_ref[j, pl.ds(0, N)] = xw[:, j * HH:(j + 1) * HH]
        xw_ref[j, pl.ds(N, N_PAD - N)] = jnp.zeros((N_PAD - N, HH), jnp.float32)


def _mm1(x, W1, deg1):
    return pl.pallas_call(
        _mm1_body,
        out_shape=[
            jax.ShapeDtypeStruct((2, N_PAD, HH), jnp.float32),
            jax.ShapeDtypeStruct((N_PAD, 1), jnp.float32),
        ],
    )(x, W1, deg1)


def _mm2_body(h_ref, w_ref, deg_ref, xw_ref, dis_ref):
    deg = deg_ref[...]
    dis = jnp.where(deg > 0, lax.rsqrt(jnp.maximum(deg, 1e-12)), 0.0)
    dis_ref[...] = dis
    w = w_ref[...]
    xw = jnp.dot(h_ref[0], w[:HH], preferred_element_type=jnp.float32)
    xw = xw + jnp.dot(h_ref[1], w[HH:], preferred_element_type=jnp.float32)
    xw = xw * dis
    for j in range(2):
        xw_ref[j] = xw[:, j * HH:(j + 1) * HH]


def _mm2(h, W2, deg2):
    return pl.pallas_call(
        _mm2_body,
        out_shape=[
            jax.ShapeDtypeStruct((2, N_PAD, HH), jnp.float32),
            jax.ShapeDtypeStruct((N_PAD, 1), jnp.float32),
        ],
    )(h, W2, deg2)


def _layer_body(scale, pool, *refs):
    if scale and pool:
        raise NotImplementedError
    if scale:
        (xw_hbm, src4d, dst4d, w4d, dis3d, bias2, h_out,
         acc_sp, src_v, dst_v, w_v, rows_a, rows_b, idx_a, idx_b,
         sidx_a, sidx_b, dis_v, bias_v, gsem_a, gsem_b, ssem_a, ssem_b) = refs
        pooled_sp = batch_v = pooled_out = None
    else:
        (xw_hbm, src4d, dst4d, dis3d, bias2, batch3d, pooled_out,
         acc_sp, pooled_sp, src_v, dst_v, rows_a, rows_b, idx_a, idx_b,
         sidx_a, sidx_b, dis_v, bias_v, batch_v,
         gsem_a, gsem_b, ssem_a, ssem_b) = refs
        w_v = None

    c = lax.axis_index("c")
    s = lax.axis_index("s")
    hbuf = rows_a.at[pl.ds(0, ERW)]
    bufs = ((rows_a, idx_a, sidx_a, gsem_a, ssem_a),
            (rows_b, idx_b, sidx_b, gsem_b, ssem_b))
    dummy_src = xw_hbm.at[pl.ds(0, ECH)]

    _zero_rows(rows_a, ECH)
    _zero_rows(rows_b, ECH)
    for jj in range(NEC):
        pltpu.sync_copy(hbuf, acc_sp.at[pl.ds(s * NPT + jj * ERW, ERW)])
    if pool:
        @pl.when(s < G_PAD // 8)
        def _():
            pltpu.sync_copy(rows_a.at[pl.ds(0, 8)], pooled_sp.at[pl.ds(s * 8, 8)])
    plsc.subcore_barrier()

    pltpu.sync_copy(dis3d.at[s], dis_v)
    pltpu.sync_copy(bias2.at[c], bias_v)
    if pool:
        pltpu.sync_copy(batch3d.at[s], batch_v)

    row_base = c * N_PAD

    for rows, _idx, sidx, _gs, ssem in bufs:
        for r in range(ECH // 16):
            sidx[pl.ds(r * 16, 16)] = lax.iota(jnp.int32, 16) + (s * NPT + r * 16)
        pltpu.async_copy(rows, acc_sp.at[sidx], ssem, add=True)

    def block(o, _):
        pltpu.sync_copy(src4d.at[s, pl.ds(o * SPB, SPB)], src_v)
        pltpu.sync_copy(dst4d.at[s, pl.ds(o * SPB, SPB)], dst_v)
        if scale:
            pltpu.sync_copy(w4d.at[s, pl.ds(o * SPB, SPB)], w_v)

        def super_chunk(gg, _):
            gd = []
            for k, (rows, idx, sidx, gsem, ssem) in enumerate(bufs):
                pltpu.make_async_copy(dummy_src, rows, ssem).wait()
                for r in range(ECH // 16):
                    idx[pl.ds(r * 16, 16)] = (
                        src_v[gg, k, pl.ds(r * 16, 16)] + row_base)
                gd.append(pltpu.async_copy(xw_hbm.at[idx], rows, gsem))
            for k, (rows, idx, sidx, gsem, ssem) in enumerate(bufs):
                gd[k].wait()
                if scale:
                    def sc16(ee, _, k=k, rows=rows):
                        wv = w_v[gg, k, pl.ds(ee * 16, 16)]
                        for kk in range(16):
                            e = ee * 16 + kk
                            w = wv[kk]
                            for r in range(HH // 16):
                                rows[e, pl.ds(r * 16, 16)] = (
                                    rows[e, pl.ds(r * 16, 16)] * w)
                        return 0

                    lax.fori_loop(0, ECH // 16, sc16, 0)
                for r in range(ECH // 16):
                    sidx[pl.ds(r * 16, 16)] = dst_v[gg, k, pl.ds(r * 16, 16)]
                pltpu.async_copy(rows, acc_sp.at[sidx], ssem, add=True)
            return 0

        lax.fori_loop(0, SPB, super_chunk, 0)
        return 0

    lax.fori_loop(0, NOB, block, 0)
    for rows, _idx, _sidx, _gs, ssem in bufs:
        pltpu.make_async_copy(dummy_src, rows, ssem).wait()
    plsc.subcore_barrier()

    def epi(jj, _):
        pltpu.sync_copy(acc_sp.at[pl.ds(s * NPT + jj * ERW, ERW)], hbuf)

        def row16(ee, _):
            dv = dis_v[jj, pl.ds(ee * 16, 16)]
            for k in range(16):
                e = ee * 16 + k
                d = dv[k]
                for r in range(HH // 16):
                    v = rows_a[e, pl.ds(r * 16, 16)] * d + bias_v[pl.ds(r * 16, 16)]
                    rows_a[e, pl.ds(r * 16, 16)] = jnp.maximum(v, 0.0)
            return 0

        lax.fori_loop(0, ERW // 16, row16, 0)
        if pool:
            pltpu.sync_copy(hbuf, pooled_sp.at[batch_v.at[jj]], add=True)
        else:
            pltpu.sync_copy(hbuf, h_out.at[c, pl.ds(s * NPT + jj * ERW, ERW)])
        return 0

    lax.fori_loop(0, NEC, epi, 0)

    if pool:
        plsc.subcore_barrier()

        @pl.when(s < G // 8)
        def _():
            pltpu.sync_copy(pooled_sp.at[pl.ds(s * 8, 8)], rows_a.at[pl.ds(0, 8)])
            pltpu.sync_copy(rows_a.at[pl.ds(0, 8)], pooled_out.at[c, pl.ds(s * 8, 8)])


_layer1_kernel = pl.kernel(
    functools.partial(_layer_body, True, False),
    out_type=jax.ShapeDtypeStruct((2, N_PAD, HH), jnp.float32),
    mesh=_mesh,
    scratch_types=[
        pltpu.VMEM_SHARED((N_PAD, HH), jnp.float32),
        pltpu.VMEM((SPB, SCH, ECH), jnp.int32),
        pltpu.VMEM((SPB, SCH, ECH), jnp.int32),
        pltpu.VMEM((SPB, SCH, ECH), jnp.float32),
        pltpu.VMEM((ECH, HH), jnp.float32),
        pltpu.VMEM((ECH, HH), jnp.float32),
        pltpu.VMEM((ECH,), jnp.int32),
        pltpu.VMEM((ECH,), jnp.int32),
        pltpu.VMEM((ECH,), jnp.int32),
        pltpu.VMEM((ECH,), jnp.int32),
        pltpu.VMEM((NEC, ERW), jnp.float32),
        pltpu.VMEM((HH,), jnp.float32),
        pltpu.SemaphoreType.DMA,
        pltpu.SemaphoreType.DMA,
        pltpu.SemaphoreType.DMA,
        pltpu.SemaphoreType.DMA,
    ],
)

_layer2_kernel = pl.kernel(
    functools.partial(_layer_body, False, True),
    out_type=jax.ShapeDtypeStruct((2, G, HH), jnp.float32),
    mesh=_mesh,
    scratch_types=[
        pltpu.VMEM_SHARED((N_PAD, HH), jnp.float32),
        pltpu.VMEM_SHARED((G_PAD, HH), jnp.float32),
        pltpu.VMEM((SPB, SCH, ECH), jnp.int32),
        pltpu.VMEM((SPB, SCH, ECH), jnp.int32),
        pltpu.VMEM((ECH, HH), jnp.float32),
        pltpu.VMEM((ECH, HH), jnp.float32),
        pltpu.VMEM((ECH,), jnp.int32),
        pltpu.VMEM((ECH,), jnp.int32),
        pltpu.VMEM((ECH,), jnp.int32),
        pltpu.VMEM((ECH,), jnp.int32),
        pltpu.VMEM((NEC, ERW), jnp.float32),
        pltpu.VMEM((HH,), jnp.float32),
        pltpu.VMEM((NEC, ERW), jnp.int32),
        pltpu.SemaphoreType.DMA,
        pltpu.SemaphoreType.DMA,
        pltpu.SemaphoreType.DMA,
        pltpu.SemaphoreType.DMA,
    ],
)


def _final_body(pooled_ref, batch_ref, wfc_ref, bfc_ref, out_ref):
    b2d = batch_ref[...]
    gids = lax.broadcasted_iota(jnp.int32, (G, N_PAD // 128, 128), 0)
    eq = (b2d[None, :, :] == gids).astype(jnp.float32)
    cnt = jnp.sum(eq, axis=(1, 2))
    cnt = jnp.maximum(cnt, 1.0)[:, None]
    pa = pooled_ref[0] / cnt
    pb = pooled_ref[1] / cnt
    wfc = wfc_ref[...]
    z = jnp.dot(pa, wfc[:HH], preferred_element_type=jnp.float32)
    z = z + jnp.dot(pb, wfc[HH:], preferred_element_type=jnp.float32)
    z = z + bfc_ref[...]
    m = jnp.max(z, axis=1, keepdims=True)
    e = jnp.exp(z - m)
    out_ref[...] = z - m - jnp.log(jnp.sum(e, axis=1, keepdims=True))


def _final(pooled, batch2d, Wfc, bfc):
    return pl.pallas_call(
        _final_body,
        out_shape=jax.ShapeDtypeStruct((G, C), jnp.float32),
    )(pooled, batch2d, Wfc, bfc.reshape(1, C))


def kernel(x, edge_index, edge_weight, batch, W1, b1, W2, b2, Wfc, bfc):
    f32 = jnp.float32
    i32 = jnp.int32
    loop = jnp.arange(N_PAD, dtype=i32)
    extra = E_PAD - E - N_PAD
    tail = jnp.full((extra,), N_PAD - 1, i32)
    sl_w = (loop < N).astype(f32)
    zw = jnp.zeros((extra,), f32)

    src_flat = jnp.concatenate([edge_index[0], loop, tail])
    dst_flat = jnp.concatenate([edge_index[1], loop, tail])
    w1_flat = jnp.concatenate([edge_weight, sl_w, zw])
    w2_flat = jnp.concatenate([jnp.ones((E,), f32), sl_w, zw])
    src4d = src_flat.reshape(NT, NOB * SPB, SCH, ECH)
    dst4d = dst_flat.reshape(NT, NOB * SPB, SCH, ECH)
    w1_4d = w1_flat.reshape(NT, NOB * SPB, SCH, ECH)
    dst3d = dst_flat.reshape(NT, NCH, ECH)
    ew_both = jnp.stack([w1_flat.reshape(NT, NCH, ECH),
                         w2_flat.reshape(NT, NCH, ECH)])

    batch_pad = jnp.concatenate(
        [batch.astype(i32), jnp.full((N_PAD - N,), G, i32)])
    batch3d = batch_pad.reshape(NT, NEC, ERW)
    batch2d = batch_pad.reshape(N_PAD // 128, 128)

    deg_both = _deg_kernel(dst3d, ew_both)
    deg1 = deg_both[0].reshape(N_PAD, 1)
    deg2 = deg_both[1].reshape(N_PAD, 1)

    xw1, dis1 = _mm1(x, W1, deg1)
    h1 = _layer1_kernel(
        xw1.reshape(2 * N_PAD, HH), src4d, dst4d, w1_4d,
        dis1.reshape(NT, NEC, ERW), b1.reshape(2, HH))

    xw2, dis2 = _mm2(h1, W2, deg2)
    pooled = _layer2_kernel(
        xw2.reshape(2 * N_PAD, HH), src4d, dst4d,
        dis2.reshape(NT, NEC, ERW), b2.reshape(2, HH), batch3d)

    return _final(pooled, batch2d, Wfc, bfc)


# speedup vs baseline: 1.0342x; 1.0025x over previous
import functools

import jax
import jax.numpy as jnp
from jax import lax
from jax.experimental import pallas as pl
from jax.experimental.pallas import tpu as pltpu
from jax.experimental.pallas import tpu_sc as plsc

N = 10000
E = 320000
F_IN = 128
H = 256
HH = 128
C = 16
G = 64
G_PAD = 72

N_PAD = 10240
NT = 16
ECH = 128
NCH = 162
EPT = NCH * ECH
E_PAD = EPT * NT
SCH = 2
SPB = 9
NOB = NCH // (SCH * SPB)
NPT = N_PAD // NT
ERW = 64
NEC = NPT // ERW

_mesh = plsc.VectorSubcoreMesh(core_axis_name="c", subcore_axis_name="s")


def _zero_rows(buf, nrows):
    zero = jnp.zeros((16,), jnp.float32)

    def body(e, _):
        for r in range(HH // 16):
            buf[e, pl.ds(r * 16, 16)] = zero
        return 0

    lax.fori_loop(0, nrows, body, 0)


def _deg_body(dst3d, w1_3d, w2_3d, deg1_out, deg2_out, deg_sp, dst_v, ew_v, buf_v):
    c = lax.axis_index("c")
    s = lax.axis_index("s")

    def zb(e, _):
        buf_v[pl.ds(e * 16, 16)] = jnp.zeros((16,), jnp.float32)
        return 0

    lax.fori_loop(0, NPT // 16, zb, 0)
    pltpu.sync_copy(buf_v, deg_sp.at[pl.ds(s * NPT, NPT)])
    plsc.subcore_barrier()

    pltpu.sync_copy(dst3d.at[s], dst_v)

    @pl.when(c == 0)
    def _():
        pltpu.sync_copy(w1_3d.at[s], ew_v)

    @pl.when(c == 1)
    def _():
        pltpu.sync_copy(w2_3d.at[s], ew_v)

    def chunk(j, _):
        pltpu.sync_copy(ew_v.at[j], deg_sp.at[dst_v.at[j]], add=True)
        return 0

    lax.fori_loop(0, NCH, chunk, 0)
    plsc.subcore_barrier()

    pltpu.sync_copy(deg_sp.at[pl.ds(s * NPT, NPT)], buf_v)

    @pl.when(c == 0)
    def _():
        pltpu.sync_copy(buf_v, deg1_out.at[pl.ds(s * NPT, NPT)])

    @pl.when(c == 1)
    def _():
        pltpu.sync_copy(buf_v, deg2_out.at[pl.ds(s * NPT, NPT)])


_deg_kernel = pl.kernel(
    _deg_body,
    out_type=[jax.ShapeDtypeStruct((N_PAD,), jnp.float32),
              jax.ShapeDtypeStruct((N_PAD,), jnp.float32)],
    mesh=_mesh,
    scratch_types=[
        pltpu.VMEM_SHARED((N_PAD,), jnp.float32),
        pltpu.VMEM((NCH, ECH), jnp.int32),
        pltpu.VMEM((NCH, ECH), jnp.float32),
        pltpu.VMEM((NPT,), jnp.float32),
    ],
)


def _mm1_body(x_ref, w_ref, deg_ref, xw_ref, dis_ref):
    deg = deg_ref[...]
    dis = jnp.where(deg > 0, lax.rsqrt(jnp.maximum(deg, 1e-12)), 0.0)
    dis_ref[...] = dis
    xw = jnp.dot(x_ref[...], w_ref[...], preferred_element_type=jnp.float32)
    xw = xw * dis[:N]
    for j in range(2):
        xw_ref[pl.ds(j * N_PAD, N)] = xw[:, j * HH:(j + 1) * HH]
        xw_ref[pl.ds(j * N_PAD + N, N_PAD - N)] = jnp.zeros(
            (N_PAD - N, HH), jnp.float32)


def _mm1(x, W1, deg1):
    return pl.pallas_call(
        _mm1_body,
        out_shape=[
            jax.ShapeDtypeStruct((2 * N_PAD, HH), jnp.float32),
            jax.ShapeDtypeStruct((N_PAD, 1), jnp.float32),
        ],
    )(x, W1, deg1)


def _mm2_body(h_ref, w_ref, deg_ref, xw_ref, dis_ref):
    deg = deg_ref[...]
    dis = jnp.where(deg > 0, lax.rsqrt(jnp.maximum(deg, 1e-12)), 0.0)
    dis_ref[...] = dis
    w = w_ref[...]
    xw = jnp.dot(h_ref[pl.ds(0, N_PAD)], w[:HH],
                 preferred_element_type=jnp.float32)
    xw = xw + jnp.dot(h_ref[pl.ds(N_PAD, N_PAD)], w[HH:],
                      preferred_element_type=jnp.float32)
    xw = xw * dis
    for j in range(2):
        xw_ref[pl.ds(j * N_PAD, N_PAD)] = xw[:, j * HH:(j + 1) * HH]


def _mm2(h, W2, deg2):
    return pl.pallas_call(
        _mm2_body,
        out_shape=[
            jax.ShapeDtypeStruct((2 * N_PAD, HH), jnp.float32),
            jax.ShapeDtypeStruct((N_PAD, 1), jnp.float32),
        ],
    )(h, W2, deg2)


def _layer_body(scale, pool, *refs):
    if scale and pool:
        raise NotImplementedError
    if scale:
        (xw_hbm, src4d, dst4d, w4d, dis3d, bias2, h_out,
         acc_sp, src_v, dst_v, w_v, rows_a, rows_b, idx_a, idx_b,
         sidx_a, sidx_b, dis_v, bias_v, gsem_a, gsem_b, ssem_a, ssem_b) = refs
        pooled_sp = batch_v = pooled_out = None
    else:
        (xw_hbm, src4d, dst4d, dis3d, bias2, batch3d, pooled_out,
         acc_sp, pooled_sp, src_v, dst_v, rows_a, rows_b, idx_a, idx_b,
         sidx_a, sidx_b, dis_v, bias_v, batch_v,
         gsem_a, gsem_b, ssem_a, ssem_b) = refs
        w_v = None

    c = lax.axis_index("c")
    s = lax.axis_index("s")
    hbuf = rows_a.at[pl.ds(0, ERW)]
    bufs = ((rows_a, idx_a, sidx_a, gsem_a, ssem_a),
            (rows_b, idx_b, sidx_b, gsem_b, ssem_b))
    dummy_src = xw_hbm.at[pl.ds(0, ECH)]

    _zero_rows(rows_a, ECH)
    _zero_rows(rows_b, ECH)
    for jj in range(NEC):
        pltpu.sync_copy(hbuf, acc_sp.at[pl.ds(s * NPT + jj * ERW, ERW)])
    if pool:
        @pl.when(s < G_PAD // 8)
        def _():
            pltpu.sync_copy(rows_a.at[pl.ds(0, 8)], pooled_sp.at[pl.ds(s * 8, 8)])
    plsc.subcore_barrier()

    pltpu.sync_copy(dis3d.at[s], dis_v)
    pltpu.sync_copy(bias2.at[c], bias_v)
    if pool:
        pltpu.sync_copy(batch3d.at[s], batch_v)

    row_base = c * N_PAD

    for rows, _idx, sidx, _gs, ssem in bufs:
        for r in range(ECH // 16):
            sidx[pl.ds(r * 16, 16)] = lax.iota(jnp.int32, 16) + (s * NPT + r * 16)
        pltpu.async_copy(rows, acc_sp.at[sidx], ssem, add=True)

    def block(o, _):
        pltpu.sync_copy(src4d.at[s, pl.ds(o * SPB, SPB)], src_v)
        pltpu.sync_copy(dst4d.at[s, pl.ds(o * SPB, SPB)], dst_v)
        if scale:
            pltpu.sync_copy(w4d.at[s, pl.ds(o * SPB, SPB)], w_v)

        def super_chunk(gg, _):
            gd = []
            for k, (rows, idx, sidx, gsem, ssem) in enumerate(bufs):
                pltpu.make_async_copy(dummy_src, rows, ssem).wait()
                for r in range(ECH // 16):
                    idx[pl.ds(r * 16, 16)] = (
                        src_v[gg, k, pl.ds(r * 16, 16)] + row_base)
                gd.append(pltpu.async_copy(xw_hbm.at[idx], rows, gsem))
            for k, (rows, idx, sidx, gsem, ssem) in enumerate(bufs):
                gd[k].wait()
                if scale:
                    def sc16(ee, _, k=k, rows=rows):
                        wv = w_v[gg, k, pl.ds(ee * 16, 16)]
                        for kk in range(16):
                            e = ee * 16 + kk
                            w = wv[kk]
                            for r in range(HH // 16):
                                rows[e, pl.ds(r * 16, 16)] = (
                                    rows[e, pl.ds(r * 16, 16)] * w)
                        return 0

                    lax.fori_loop(0, ECH // 16, sc16, 0)
                for r in range(ECH // 16):
                    sidx[pl.ds(r * 16, 16)] = dst_v[gg, k, pl.ds(r * 16, 16)]
                pltpu.async_copy(rows, acc_sp.at[sidx], ssem, add=True)
            return 0

        lax.fori_loop(0, SPB, super_chunk, 0)
        return 0

    lax.fori_loop(0, NOB, block, 0)
    for rows, _idx, _sidx, _gs, ssem in bufs:
        pltpu.make_async_copy(dummy_src, rows, ssem).wait()
    plsc.subcore_barrier()

    def epi(jj, _):
        pltpu.sync_copy(acc_sp.at[pl.ds(s * NPT + jj * ERW, ERW)], hbuf)

        def row16(ee, _):
            dv = dis_v[jj, pl.ds(ee * 16, 16)]
            for k in range(16):
                e = ee * 16 + k
                d = dv[k]
                for r in range(HH // 16):
                    v = rows_a[e, pl.ds(r * 16, 16)] * d + bias_v[pl.ds(r * 16, 16)]
                    rows_a[e, pl.ds(r * 16, 16)] = jnp.maximum(v, 0.0)
            return 0

        lax.fori_loop(0, ERW // 16, row16, 0)
        if pool:
            pltpu.sync_copy(hbuf, pooled_sp.at[batch_v.at[jj]], add=True)
        else:
            pltpu.sync_copy(
                hbuf, h_out.at[pl.ds(c * N_PAD + s * NPT + jj * ERW, ERW)])
        return 0

    lax.fori_loop(0, NEC, epi, 0)

    if pool:
        plsc.subcore_barrier()

        @pl.when(s < G // 8)
        def _():
            pltpu.sync_copy(pooled_sp.at[pl.ds(s * 8, 8)], rows_a.at[pl.ds(0, 8)])
            pltpu.sync_copy(rows_a.at[pl.ds(0, 8)], pooled_out.at[c, pl.ds(s * 8, 8)])


_layer1_kernel = pl.kernel(
    functools.partial(_layer_body, True, False),
    out_type=jax.ShapeDtypeStruct((2 * N_PAD, HH), jnp.float32),
    mesh=_mesh,
    scratch_types=[
        pltpu.VMEM_SHARED((N_PAD, HH), jnp.float32),
        pltpu.VMEM((SPB, SCH, ECH), jnp.int32),
        pltpu.VMEM((SPB, SCH, ECH), jnp.int32),
        pltpu.VMEM((SPB, SCH, ECH), jnp.float32),
        pltpu.VMEM((ECH, HH), jnp.float32),
        pltpu.VMEM((ECH, HH), jnp.float32),
        pltpu.VMEM((ECH,), jnp.int32),
        pltpu.VMEM((ECH,), jnp.int32),
        pltpu.VMEM((ECH,), jnp.int32),
        pltpu.VMEM((ECH,), jnp.int32),
        pltpu.VMEM((NEC, ERW), jnp.float32),
        pltpu.VMEM((HH,), jnp.float32),
        pltpu.SemaphoreType.DMA,
        pltpu.SemaphoreType.DMA,
        pltpu.SemaphoreType.DMA,
        pltpu.SemaphoreType.DMA,
    ],
)

_layer2_kernel = pl.kernel(
    functools.partial(_layer_body, False, True),
    out_type=jax.ShapeDtypeStruct((2, G, HH), jnp.float32),
    mesh=_mesh,
    scratch_types=[
        pltpu.VMEM_SHARED((N_PAD, HH), jnp.float32),
        pltpu.VMEM_SHARED((G_PAD, HH), jnp.float32),
        pltpu.VMEM((SPB, SCH, ECH), jnp.int32),
        pltpu.VMEM((SPB, SCH, ECH), jnp.int32),
        pltpu.VMEM((ECH, HH), jnp.float32),
        pltpu.VMEM((ECH, HH), jnp.float32),
        pltpu.VMEM((ECH,), jnp.int32),
        pltpu.VMEM((ECH,), jnp.int32),
        pltpu.VMEM((ECH,), jnp.int32),
        pltpu.VMEM((ECH,), jnp.int32),
        pltpu.VMEM((NEC, ERW), jnp.float32),
        pltpu.VMEM((HH,), jnp.float32),
        pltpu.VMEM((NEC, ERW), jnp.int32),
        pltpu.SemaphoreType.DMA,
        pltpu.SemaphoreType.DMA,
        pltpu.SemaphoreType.DMA,
        pltpu.SemaphoreType.DMA,
    ],
)


def _final_body(pooled_ref, batch_ref, wfc_ref, bfc_ref, out_ref):
    b2d = batch_ref[...]
    gids = lax.broadcasted_iota(jnp.int32, (G, N_PAD // 128, 128), 0)
    eq = (b2d[None, :, :] == gids).astype(jnp.float32)
    cnt = jnp.sum(eq, axis=(1, 2))
    cnt = jnp.maximum(cnt, 1.0)[:, None]
    pa = pooled_ref[0] / cnt
    pb = pooled_ref[1] / cnt
    wfc = wfc_ref[...]
    z = jnp.dot(pa, wfc[:HH], preferred_element_type=jnp.float32)
    z = z + jnp.dot(pb, wfc[HH:], preferred_element_type=jnp.float32)
    z = z + bfc_ref[...]
    m = jnp.max(z, axis=1, keepdims=True)
    e = jnp.exp(z - m)
    out_ref[...] = z - m - jnp.log(jnp.sum(e, axis=1, keepdims=True))


def _final(pooled, batch2d, Wfc, bfc):
    return pl.pallas_call(
        _final_body,
        out_shape=jax.ShapeDtypeStruct((G, C), jnp.float32),
    )(pooled, batch2d, Wfc, bfc.reshape(1, C))


def kernel(x, edge_index, edge_weight, batch, W1, b1, W2, b2, Wfc, bfc):
    f32 = jnp.float32
    i32 = jnp.int32
    loop = jnp.arange(N_PAD, dtype=i32)
    extra = E_PAD - E - N_PAD
    tail = jnp.full((extra,), N_PAD - 1, i32)
    sl_w = (loop < N).astype(f32)
    zw = jnp.zeros((extra,), f32)

    src_flat = jnp.concatenate([edge_index[0], loop, tail])
    dst_flat = jnp.concatenate([edge_index[1], loop, tail])
    w1_flat = jnp.concatenate([edge_weight, sl_w, zw])
    w2_flat = jnp.concatenate([jnp.ones((E,), f32), sl_w, zw])
    src4d = src_flat.reshape(NT, NOB * SPB, SCH, ECH)
    dst4d = dst_flat.reshape(NT, NOB * SPB, SCH, ECH)
    w1_4d = w1_flat.reshape(NT, NOB * SPB, SCH, ECH)
    dst3d = dst_flat.reshape(NT, NCH, ECH)
    w1_3d = w1_flat.reshape(NT, NCH, ECH)
    w2_3d = w2_flat.reshape(NT, NCH, ECH)

    batch_pad = jnp.concatenate(
        [batch.astype(i32), jnp.full((N_PAD - N,), G, i32)])
    batch3d = batch_pad.reshape(NT, NEC, ERW)
    batch2d = batch_pad.reshape(N_PAD // 128, 128)

    deg1, deg2 = _deg_kernel(dst3d, w1_3d, w2_3d)

    xw1, dis1 = _mm1(x, W1, deg1.reshape(N_PAD, 1))
    h1 = _layer1_kernel(
        xw1, src4d, dst4d, w1_4d,
        dis1.reshape(NT, NEC, ERW), b1.reshape(2, HH))

    xw2, dis2 = _mm2(h1, W2, deg2.reshape(N_PAD, 1))
    pooled = _layer2_kernel(
        xw2, src4d, dst4d,
        dis2.reshape(NT, NEC, ERW), b2.reshape(2, HH), batch3d)

    return _final(pooled, batch2d, Wfc, bfc)
